# Initial kernel scaffold; baseline (speedup 1.0000x reference)
import jax
import jax.numpy as jnp
from jax.experimental import pallas as pl


def kernel(x_conversation, x_sentence, x_word, edge_cs, edge_ss, edge_sw, edge_ww, edge_sc, edge_ws, W_conv, b_conv, W_sent, b_sent, W_word, b_word, Wl_cs, bl_cs, Wr_cs, Wl_ss, bl_ss, Wr_ss, Wl_sw, bl_sw, Wr_sw, Wl_ww, bl_ww, Wr_ww, Wl_sc, bl_sc, Wr_sc, Wl_ws, bl_ws, Wr_ws):
    raise NotImplementedError("write your pallas kernel here")


# speedup vs baseline: 1.6009x; 1.6009x over previous
import functools

import jax
import jax.numpy as jnp
from jax import lax
from jax.experimental import pallas as pl
from jax.experimental.pallas import tpu as pltpu
from jax.experimental.pallas import tpu_sc as plsc

H = 256


def _proj_body(x_ref, w_ref, b_ref, o_ref):
    o_ref[...] = lax.dot_general(
        x_ref[...], w_ref[...], (((1,), (1,)), ((), ()))) + b_ref[...]


def _project(x, w, b, bm):
    m, k = x.shape
    return pl.pallas_call(
        _proj_body,
        grid=(m // bm,),
        in_specs=[
            pl.BlockSpec((bm, k), lambda i: (i, 0)),
            pl.BlockSpec((H, k), lambda i: (0, 0)),
            pl.BlockSpec((1, H), lambda i: (0, 0)),
        ],
        out_specs=pl.BlockSpec((bm, H), lambda i: (i, 0)),
        out_shape=jax.ShapeDtypeStruct((m, H), jnp.float32),
    )(x, w, b.reshape(1, H))


def _make_seg(e_pad, npad):
    ept = e_pad // 32
    nbt = ept // 128
    stripe = npad // 16
    mesh = plsc.VectorSubcoreMesh(core_axis_name="c", subcore_axis_name="s")

    @functools.partial(
        pl.kernel,
        mesh=mesh,
        compiler_params=pltpu.CompilerParams(needs_layout_passes=False),
        out_type=[
            jax.ShapeDtypeStruct((npad, H), jnp.float32),
            jax.ShapeDtypeStruct((npad, H), jnp.float32),
            jax.ShapeDtypeStruct((npad, H), jnp.float32),
            jax.ShapeDtypeStruct((npad, H), jnp.float32),
        ],
        scratch_types=[
            pltpu.VMEM((nbt, 128), jnp.int32),
            pltpu.VMEM((nbt, 128), jnp.int32),
            pltpu.VMEM((128, H), jnp.float32),
            pltpu.VMEM((128, H), jnp.float32),
            pltpu.VMEM((128, H), jnp.float32),
            pltpu.SemaphoreType.DMA,
        ],
    )
    def seg(h_hbm, src_hbm, dst_hbm, zr_hbm, on_hbm,
            ssum0, cnt0, ssum1, cnt1, srcv, dstv, stage, onesv, zrv, sem):
        cid = lax.axis_index("c")
        sid = lax.axis_index("s")
        wid = cid * 16 + sid
        for g in range(nbt):
            pltpu.sync_copy(src_hbm.at[pl.ds(wid * ept + g * 128, 128)],
                            srcv.at[g])
            pltpu.sync_copy(dst_hbm.at[pl.ds(wid * ept + g * 128, 128)],
                            dstv.at[g])
        pltpu.sync_copy(on_hbm, onesv)
        pltpu.sync_copy(zr_hbm, zrv)

        def run(ssum, cnt):
            for off in range(0, stripe, 128):
                sz = min(128, stripe - off)
                r0 = sid * stripe + off
                pltpu.sync_copy(zrv.at[pl.ds(0, sz)], ssum.at[pl.ds(r0, sz)])
            for off in range(0, stripe, 128):
                sz = min(128, stripe - off)
                r0 = sid * stripe + off
                pltpu.sync_copy(zrv.at[pl.ds(0, sz)], cnt.at[pl.ds(r0, sz)])
            plsc.subcore_barrier()

            def batch(g, c):
                si = srcv.at[g]
                di = dstv.at[g]
                pltpu.async_copy(h_hbm.at[si], stage, sem).wait()
                pltpu.async_copy(stage, ssum.at[di], sem, add=True).wait()
                pltpu.async_copy(onesv, cnt.at[di], sem, add=True).wait()
                return c

            lax.fori_loop(0, nbt, batch, jnp.int32(0))

        @pl.when(cid == 0)
        def _sc0():
            run(ssum0, cnt0)

        @pl.when(cid == 1)
        def _sc1():
            run(ssum1, cnt1)

    return seg


def _segment_parts(h_src, edge, npad, zr, on):
    e = edge.shape[1]
    e_pad = -(-e // 4096) * 4096
    src = jnp.concatenate(
        [edge[0].astype(jnp.int32), jnp.zeros((e_pad - e,), jnp.int32)])
    dst = jnp.concatenate(
        [edge[1].astype(jnp.int32),
         jnp.full((e_pad - e,), npad - 1, jnp.int32)])
    return _make_seg(e_pad, npad)(h_src, src, dst, zr, on)


def _combine(h_dst, parts, wls, wr_sum, bl_sum, bm=1000):
    n = h_dst.shape[0]
    r = len(parts)

    def body(*refs):
        h_ref = refs[0]
        out_ref = refs[-1]
        wr_ref = refs[1 + 5 * r]
        b_ref = refs[2 + 5 * r]
        acc = lax.dot_general(
            h_ref[...], wr_ref[...], (((1,), (1,)), ((), ())))
        for j in range(r):
            s0 = refs[1 + 5 * j][...]
            c0 = refs[2 + 5 * j][...]
            s1 = refs[3 + 5 * j][...]
            c1 = refs[4 + 5 * j][...]
            wl = refs[5 + 5 * j][...]
            mean = (s0 + s1) / jnp.maximum(c0[:, 0:1] + c1[:, 0:1], 1.0)
            acc += lax.dot_general(mean, wl, (((1,), (1,)), ((), ())))
        out_ref[...] = acc + b_ref[...]

    in_specs = [pl.BlockSpec((bm, H), lambda i: (i, 0))]
    args = [h_dst]
    for (s0, c0, s1, c1), wl in zip(parts, wls):
        in_specs.append(pl.BlockSpec((bm, H), lambda i: (i, 0)))
        in_specs.append(pl.BlockSpec((bm, H), lambda i: (i, 0)))
        in_specs.append(pl.BlockSpec((bm, H), lambda i: (i, 0)))
        in_specs.append(pl.BlockSpec((bm, H), lambda i: (i, 0)))
        in_specs.append(pl.BlockSpec((H, H), lambda i: (0, 0)))
        args += [s0, c0, s1, c1, wl]
    in_specs.append(pl.BlockSpec((H, H), lambda i: (0, 0)))
    in_specs.append(pl.BlockSpec((1, H), lambda i: (0, 0)))
    args += [wr_sum, bl_sum.reshape(1, H)]
    return pl.pallas_call(
        body,
        grid=(n // bm,),
        in_specs=in_specs,
        out_specs=pl.BlockSpec((bm, H), lambda i: (i, 0)),
        out_shape=jax.ShapeDtypeStruct((n, H), jnp.float32),
    )(*args)


def _round_npad(n):
    return -(-n // 128) * 128 + 128


def kernel(x_conversation, x_sentence, x_word, edge_cs, edge_ss, edge_sw,
           edge_ww, edge_sc, edge_ws, W_conv, b_conv, W_sent, b_sent,
           W_word, b_word, Wl_cs, bl_cs, Wr_cs, Wl_ss, bl_ss, Wr_ss,
           Wl_sw, bl_sw, Wr_sw, Wl_ww, bl_ww, Wr_ww, Wl_sc, bl_sc, Wr_sc,
           Wl_ws, bl_ws, Wr_ws):
    hc = _project(x_conversation, W_conv, b_conv, bm=1000)
    hs = _project(x_sentence, W_sent, b_sent, bm=1000)
    hw = _project(x_word, W_word, b_word, bm=1000)

    np_c = _round_npad(x_conversation.shape[0])
    np_s = _round_npad(x_sentence.shape[0])
    np_w = _round_npad(x_word.shape[0])

    zr = jnp.zeros((128, H), jnp.float32)
    on = jnp.ones((128, H), jnp.float32)

    p_cs = _segment_parts(hc, edge_cs, np_s, zr, on)
    p_ss = _segment_parts(hs, edge_ss, np_s, zr, on)
    p_ws = _segment_parts(hw, edge_ws, np_s, zr, on)
    p_sw = _segment_parts(hs, edge_sw, np_w, zr, on)
    p_ww = _segment_parts(hw, edge_ww, np_w, zr, on)
    p_sc = _segment_parts(hs, edge_sc, np_c, zr, on)

    out_s = _combine(hs, [p_cs, p_ss, p_ws], [Wl_cs, Wl_ss, Wl_ws],
                     Wr_cs + Wr_ss + Wr_ws, bl_cs + bl_ss + bl_ws)
    out_w = _combine(hw, [p_sw, p_ww], [Wl_sw, Wl_ww],
                     Wr_sw + Wr_ww, bl_sw + bl_ww)
    out_c = _combine(hc, [p_sc], [Wl_sc], Wr_sc, bl_sc)
    return (out_c, out_s, out_w)
